# baseline (speedup 1.0000x reference)
import functools

import jax
import jax.numpy as jnp
from jax import lax
from jax.experimental import pallas as pl
from jax.experimental.pallas import tpu as pltpu
from jax.experimental.pallas import tpu_sc as plsc


def _tc_stage(x_r, e, total_count):
    B, C, HW = x_r.shape
    K = e.shape[1]
    inv_count = 1.0 / total_count

    def body(x_ref, e_ref, idx_ref, loss_ref, tab_ref, acc_ref):
        i = pl.program_id(0)
        xb = x_ref[0]
        et = e_ref[...]
        col0 = pl.multiple_of((i // 2) * 128, 128)
        tab_ref[...] = e_ref[:, pl.ds(col0, 128)].T
        x2 = jnp.sum(xb * xb, axis=0)
        e2 = jnp.sum(et * et, axis=0)
        xe_t = lax.dot_general(
            et, xb, (((0,), (0,)), ((), ())),
            preferred_element_type=jnp.float32)
        scores_t = (x2[None, :] - 2.0 * xe_t) + e2[:, None]
        mins = jnp.min(scores_t, axis=0)
        idx = jnp.argmin(scores_t, axis=0).astype(jnp.int32)
        idx_ref[0, 0, :] = idx

        @pl.when(i == 0)
        def _():
            acc_ref[...] = jnp.zeros_like(acc_ref)

        acc_ref[...] += mins.reshape(acc_ref.shape)

        @pl.when(i == pl.num_programs(0) - 1)
        def _():
            loss_ref[0, 0] = jnp.sum(acc_ref[...]) * inv_count

    return pl.pallas_call(
        body,
        grid=(B,),
        in_specs=[
            pl.BlockSpec((1, C, HW), lambda i: (i, 0, 0)),
            pl.BlockSpec((C, K), lambda i: (0, 0)),
        ],
        out_specs=[
            pl.BlockSpec((1, 1, HW), lambda i: (i, 0, 0)),
            pl.BlockSpec(block_shape=(1, 1), index_map=lambda i: (0, 0),
                         memory_space=pltpu.SMEM),
            pl.BlockSpec((128, C), lambda i: (i // 2, 0)),
        ],
        out_shape=[
            jax.ShapeDtypeStruct((B, 1, HW), jnp.int32),
            jax.ShapeDtypeStruct((1, 1), jnp.float32),
            jax.ShapeDtypeStruct((K, C), jnp.float32),
        ],
        scratch_shapes=[pltpu.VMEM((8, HW // 8), jnp.float32)],
        compiler_params=pltpu.CompilerParams(
            dimension_semantics=("arbitrary",)),
    )(x_r, e)


def _sc_gather(table, idx2d):
    K, C = table.shape
    R, CH = idx2d.shape
    info = plsc.get_sparse_core_info()
    NW = info.num_cores * info.num_subcores
    nch = R // NW
    NB = min(3, nch)

    mesh = plsc.VectorSubcoreMesh(core_axis_name="c", subcore_axis_name="s")

    @functools.partial(
        pl.kernel,
        mesh=mesh,
        out_type=jax.ShapeDtypeStruct((R * CH, C), jnp.float32),
        scratch_types=[
            pltpu.VMEM((nch, CH), jnp.int32),
        ]
        + [pltpu.VMEM((CH, C), jnp.float32) for _ in range(NB)]
        + [pltpu.SemaphoreType.DMA for _ in range(2 * NB)],
    )
    def k(table_hbm, idx_hbm, out_hbm, idx_v, *rest):
        bufs = rest[:NB]
        gsems = rest[NB:2 * NB]
        osems = rest[2 * NB:]
        wid = lax.axis_index("s") * info.num_cores + lax.axis_index("c")
        row0 = wid * nch
        pltpu.sync_copy(idx_hbm.at[pl.ds(row0, nch)], idx_v)
        gh = [None] * nch
        oh = [None] * nch
        for c in range(NB):
            gh[c] = pltpu.async_copy(table_hbm.at[idx_v.at[c]], bufs[c],
                                     gsems[c])
        for c in range(nch):
            b = c % NB
            gh[c].wait()
            oh[c] = pltpu.async_copy(
                bufs[b], out_hbm.at[pl.ds((row0 + c) * CH, CH)], osems[b])
            n = c + NB
            if n < nch:
                oh[c].wait()
                gh[n] = pltpu.async_copy(table_hbm.at[idx_v.at[n]], bufs[b],
                                         gsems[b])
        for c in range(max(0, nch - NB), nch):
            oh[c].wait()

    return k(table, idx2d)


def kernel(x, e_i_ts):
    B, C, H, W = x.shape
    HW = H * W
    x_r = x.reshape(B, C, HW)
    total = B * C * HW
    idx3, loss_arr, table = _tc_stage(x_r, e_i_ts, total)
    q_flat = _sc_gather(table, idx3.reshape(-1, 128))
    q = q_flat.reshape(B, H, W, C).transpose(0, 3, 1, 2)
    loss = loss_arr[0, 0]
    return (q, loss, loss, idx3.reshape(B, HW))


# speedup vs baseline: 1.0683x; 1.0683x over previous
import functools

import jax
import jax.numpy as jnp
from jax import lax
from jax.experimental import pallas as pl
from jax.experimental.pallas import tpu as pltpu
from jax.experimental.pallas import tpu_sc as plsc


def _tc_stage(x_r, e, total_count):
    B, C, HW = x_r.shape
    K = e.shape[1]
    inv_count = 1.0 / total_count

    def body(x_ref, e_ref, idx_ref, loss_ref, acc_ref):
        i = pl.program_id(0)
        xb = x_ref[0]
        et = e_ref[...]
        x2 = jnp.sum(xb * xb, axis=0)
        e2 = jnp.sum(et * et, axis=0)
        xe_t = lax.dot_general(
            et, xb, (((0,), (0,)), ((), ())),
            preferred_element_type=jnp.float32)
        scores_t = (x2[None, :] - 2.0 * xe_t) + e2[:, None]
        mins = jnp.min(scores_t, axis=0)
        idx = jnp.argmin(scores_t, axis=0).astype(jnp.int32)
        idx_ref[0, 0, :] = idx

        @pl.when(i == 0)
        def _():
            acc_ref[...] = jnp.zeros_like(acc_ref)

        acc_ref[...] += mins.reshape(acc_ref.shape)

        @pl.when(i == pl.num_programs(0) - 1)
        def _():
            loss_ref[0, 0] = jnp.sum(acc_ref[...]) * inv_count

    return pl.pallas_call(
        body,
        grid=(B,),
        in_specs=[
            pl.BlockSpec((1, C, HW), lambda i: (i, 0, 0)),
            pl.BlockSpec((C, K), lambda i: (0, 0)),
        ],
        out_specs=[
            pl.BlockSpec((1, 1, HW), lambda i: (i, 0, 0)),
            pl.BlockSpec(block_shape=(1, 1), index_map=lambda i: (0, 0),
                         memory_space=pltpu.SMEM),
        ],
        out_shape=[
            jax.ShapeDtypeStruct((B, 1, HW), jnp.int32),
            jax.ShapeDtypeStruct((1, 1), jnp.float32),
        ],
        scratch_shapes=[pltpu.VMEM((8, HW // 8), jnp.float32)],
        compiler_params=pltpu.CompilerParams(
            dimension_semantics=("arbitrary",)),
    )(x_r, e)


def _sc_gather(table, idx2d):
    K, C = table.shape
    R, CH = idx2d.shape
    info = plsc.get_sparse_core_info()
    NW = info.num_cores * info.num_subcores
    nch = R // NW
    NB = min(3, nch)

    mesh = plsc.VectorSubcoreMesh(core_axis_name="c", subcore_axis_name="s")

    @functools.partial(
        pl.kernel,
        mesh=mesh,
        out_type=jax.ShapeDtypeStruct((R * CH, C), jnp.float32),
        scratch_types=[
            pltpu.VMEM((nch, CH), jnp.int32),
        ]
        + [pltpu.VMEM((CH, C), jnp.float32) for _ in range(NB)]
        + [pltpu.SemaphoreType.DMA for _ in range(2 * NB)],
    )
    def k(table_hbm, idx_hbm, out_hbm, idx_v, *rest):
        bufs = rest[:NB]
        gsems = rest[NB:2 * NB]
        osems = rest[2 * NB:]
        wid = lax.axis_index("s") * info.num_cores + lax.axis_index("c")
        row0 = wid * nch
        pltpu.sync_copy(idx_hbm.at[pl.ds(row0, nch)], idx_v)
        gh = [None] * nch
        oh = [None] * nch
        for c in range(NB):
            gh[c] = pltpu.async_copy(table_hbm.at[idx_v.at[c]], bufs[c],
                                     gsems[c])
        for c in range(nch):
            b = c % NB
            gh[c].wait()
            oh[c] = pltpu.async_copy(
                bufs[b], out_hbm.at[pl.ds((row0 + c) * CH, CH)], osems[b])
            n = c + NB
            if n < nch:
                oh[c].wait()
                gh[n] = pltpu.async_copy(table_hbm.at[idx_v.at[n]], bufs[b],
                                         gsems[b])
        for c in range(max(0, nch - NB), nch):
            oh[c].wait()

    return k(table, idx2d)


def kernel(x, e_i_ts):
    B, C, H, W = x.shape
    HW = H * W
    x_r = x.reshape(B, C, HW)
    table = e_i_ts.T
    total = B * C * HW
    idx3, loss_arr = _tc_stage(x_r, e_i_ts, total)
    q_flat = _sc_gather(table, idx3.reshape(-1, 128))
    q = q_flat.reshape(B, H, W, C).transpose(0, 3, 1, 2)
    loss = loss_arr[0, 0]
    return (q, loss, loss, idx3.reshape(B, HW))


# speedup vs baseline: 1.0835x; 1.0142x over previous
import functools

import jax
import jax.numpy as jnp
from jax import lax
from jax.experimental import pallas as pl
from jax.experimental.pallas import tpu as pltpu
from jax.experimental.pallas import tpu_sc as plsc


def _tc_stage(x_r, e, total_count):
    B, C, HW = x_r.shape
    K = e.shape[1]
    inv_count = 1.0 / total_count

    def body(x_ref, e_ref, idx_ref, loss_ref, acc_ref):
        i = pl.program_id(0)
        xb = x_ref[0]
        et = e_ref[...]
        x2 = jnp.sum(xb * xb, axis=0)
        e2 = jnp.sum(et * et, axis=0)
        xe_t = lax.dot_general(
            et * -2.0, xb, (((0,), (0,)), ((), ())),
            preferred_element_type=jnp.float32)
        scores_t = (x2[None, :] + xe_t) + e2[:, None]
        mins = jnp.min(scores_t, axis=0)
        idx = jnp.argmin(scores_t, axis=0).astype(jnp.int32)
        idx_ref[0, 0, :] = idx

        @pl.when(i == 0)
        def _():
            acc_ref[...] = jnp.zeros_like(acc_ref)

        acc_ref[...] += mins.reshape(acc_ref.shape)

        @pl.when(i == pl.num_programs(0) - 1)
        def _():
            loss_ref[0, 0] = jnp.sum(acc_ref[...]) * inv_count

    return pl.pallas_call(
        body,
        grid=(B,),
        in_specs=[
            pl.BlockSpec((1, C, HW), lambda i: (i, 0, 0)),
            pl.BlockSpec((C, K), lambda i: (0, 0)),
        ],
        out_specs=[
            pl.BlockSpec((1, 1, HW), lambda i: (i, 0, 0)),
            pl.BlockSpec(block_shape=(1, 1), index_map=lambda i: (0, 0),
                         memory_space=pltpu.SMEM),
        ],
        out_shape=[
            jax.ShapeDtypeStruct((B, 1, HW), jnp.int32),
            jax.ShapeDtypeStruct((1, 1), jnp.float32),
        ],
        scratch_shapes=[pltpu.VMEM((8, HW // 8), jnp.float32)],
        compiler_params=pltpu.CompilerParams(
            dimension_semantics=("arbitrary",)),
    )(x_r, e)


def _sc_gather(table, idx2d):
    K, C = table.shape
    R, CH = idx2d.shape
    info = plsc.get_sparse_core_info()
    NW = info.num_cores * info.num_subcores
    nch = R // NW
    NB = min(3, nch)

    mesh = plsc.VectorSubcoreMesh(core_axis_name="c", subcore_axis_name="s")

    @functools.partial(
        pl.kernel,
        mesh=mesh,
        out_type=jax.ShapeDtypeStruct((R * CH, C), jnp.float32),
        scratch_types=[
            pltpu.VMEM((nch, CH), jnp.int32),
        ]
        + [pltpu.VMEM((CH, C), jnp.float32) for _ in range(NB)]
        + [pltpu.SemaphoreType.DMA for _ in range(2 * NB)],
    )
    def k(table_hbm, idx_hbm, out_hbm, idx_v, *rest):
        bufs = rest[:NB]
        gsems = rest[NB:2 * NB]
        osems = rest[2 * NB:]
        wid = lax.axis_index("s") * info.num_cores + lax.axis_index("c")
        row0 = wid * nch
        pltpu.sync_copy(idx_hbm.at[pl.ds(row0, nch)], idx_v)
        gh = [None] * nch
        oh = [None] * nch
        for c in range(NB):
            gh[c] = pltpu.async_copy(table_hbm.at[idx_v.at[c]], bufs[c],
                                     gsems[c])
        for c in range(nch):
            b = c % NB
            gh[c].wait()
            oh[c] = pltpu.async_copy(
                bufs[b], out_hbm.at[pl.ds((row0 + c) * CH, CH)], osems[b])
            n = c + NB
            if n < nch:
                oh[c].wait()
                gh[n] = pltpu.async_copy(table_hbm.at[idx_v.at[n]], bufs[b],
                                         gsems[b])
        for c in range(max(0, nch - NB), nch):
            oh[c].wait()

    return k(table, idx2d)


def kernel(x, e_i_ts):
    B, C, H, W = x.shape
    HW = H * W
    x_r = x.reshape(B, C, HW)
    table = e_i_ts.T
    total = B * C * HW
    idx3, loss_arr = _tc_stage(x_r, e_i_ts, total)
    q_flat = _sc_gather(table, idx3.reshape(-1, 128))
    q = q_flat.reshape(B, H, W, C).transpose(0, 3, 1, 2)
    loss = loss_arr[0, 0]
    return (q, loss, loss, idx3.reshape(B, HW))
